# baseline (speedup 1.0000x reference)
import functools

import jax
import jax.numpy as jnp
from jax import lax
from jax.experimental import pallas as pl
from jax.experimental.pallas import tpu as pltpu
from jax.experimental.pallas import tpu_sc as plsc

VOCAB = 100
DIM = 8
B_TOTAL = 16384 * 200

NUM_CORES = 2
NUM_SUBCORES = 16
NUM_WORKERS = NUM_CORES * NUM_SUBCORES
IDS_PER_WORKER = B_TOTAL // NUM_WORKERS
CHUNK = 2048
STEPS = IDS_PER_WORKER // CHUNK
UNROLL = 8


def _table_body(emb_ref, w_ref, b_ref, table_ref):
    table_ref[...] = (
        jnp.dot(emb_ref[...], w_ref[...].T, preferred_element_type=jnp.float32)
        + b_ref[...]
    )


_table_call = pl.pallas_call(
    _table_body,
    out_shape=jax.ShapeDtypeStruct((VOCAB, DIM), jnp.float32),
)

_sc_mesh = plsc.VectorSubcoreMesh(core_axis_name="c", subcore_axis_name="s")


@functools.partial(
    pl.kernel,
    mesh=_sc_mesh,
    compiler_params=pltpu.CompilerParams(needs_layout_passes=False),
    out_type=jax.ShapeDtypeStruct((B_TOTAL * DIM,), jnp.float32),
    scratch_types=[
        pltpu.VMEM((VOCAB * DIM,), jnp.float32),
        pltpu.VMEM((CHUNK,), jnp.int32),
        pltpu.VMEM((CHUNK,), jnp.int32),
        pltpu.VMEM((CHUNK * DIM,), jnp.float32),
        pltpu.VMEM((CHUNK * DIM,), jnp.float32),
        pltpu.SemaphoreType.DMA,
        pltpu.SemaphoreType.DMA,
        pltpu.SemaphoreType.DMA,
        pltpu.SemaphoreType.DMA,
    ],
)
def _gather_kernel(table_hbm, idx_hbm, out_hbm, table_v, idx_a, idx_b,
                   rows_a, rows_b, sem_ia, sem_ib, sem_oa, sem_ob):
    wid = lax.axis_index("s") * NUM_CORES + lax.axis_index("c")
    base = wid * IDS_PER_WORKER
    bufs = ((idx_a, rows_a, sem_ia, sem_oa), (idx_b, rows_b, sem_ib, sem_ob))

    pltpu.sync_copy(table_hbm, table_v)

    def idx_src(i):
        return idx_hbm.at[pl.ds(base + i * CHUNK, CHUNK)]

    def out_dst(i):
        return out_hbm.at[pl.ds((base + i * CHUNK) * DIM, CHUNK * DIM)]

    pltpu.async_copy(idx_src(0), bufs[0][0], bufs[0][2])
    pltpu.async_copy(idx_src(1), bufs[1][0], bufs[1][2])

    def gbody(g, carry):
        for b in range(2):
            idx_v, rows_v, sem_i, sem_o = bufs[b]
            i = 2 * g + b
            pltpu.make_async_copy(idx_src(i), idx_v, sem_i).wait()

            @pl.when(i >= 2)
            def _():
                pltpu.make_async_copy(rows_v, out_dst(i - 2), sem_o).wait()

            def cbody(u, c):
                lane = lax.iota(jnp.int32, 16)
                step8 = lane // 8
                lanemod = lane % 8
                for k in range(UNROLL):
                    v = u * UNROLL + k
                    pair = plsc.load_gather(idx_v, [2 * v + step8])
                    val = plsc.load_gather(table_v, [pair * DIM + lanemod])
                    rows_v[pl.ds(16 * v, 16)] = val
                return c

            lax.fori_loop(0, CHUNK // 2 // UNROLL, cbody, 0)

            pltpu.async_copy(rows_v, out_dst(i), sem_o)

            @pl.when(i + 2 < STEPS)
            def _():
                pltpu.async_copy(idx_src(i + 2), idx_v, sem_i)

        return carry

    lax.fori_loop(0, STEPS // 2, gbody, 0)

    pltpu.make_async_copy(rows_a, out_dst(STEPS - 2), sem_oa).wait()
    pltpu.make_async_copy(rows_b, out_dst(STEPS - 1), sem_ob).wait()


def kernel(input_ids, emb, W, b):
    table = _table_call(emb, W, b.reshape(1, DIM)).reshape(-1)
    idx = input_ids.reshape(-1).astype(jnp.int32)
    out = _gather_kernel(table, idx)
    return out.reshape(input_ids.shape + (DIM,))


# speedup vs baseline: 8.6074x; 8.6074x over previous
import functools

import jax
import jax.numpy as jnp
from jax import lax
from jax.experimental import pallas as pl
from jax.experimental.pallas import tpu as pltpu
from jax.experimental.pallas import tpu_sc as plsc

VOCAB = 100
DIM = 8
B_TOTAL = 16384 * 200

NUM_CORES = 2
NUM_SUBCORES = 16
NUM_WORKERS = NUM_CORES * NUM_SUBCORES
JH = 25
IH = 128
IH_PER_W = IH // NUM_WORKERS
CHUNK = IH_PER_W * 8 * 128
ROWS_PER_STEP = CHUNK * DIM


def _table_body(emb_ref, w_ref, b_ref, table_ref):
    table_ref[...] = (
        jnp.dot(w_ref[...], emb_ref[...].T, preferred_element_type=jnp.float32)
        + b_ref[...]
    )


_table_call = pl.pallas_call(
    _table_body,
    out_shape=jax.ShapeDtypeStruct((DIM, VOCAB), jnp.float32),
)

_sc_mesh = plsc.VectorSubcoreMesh(core_axis_name="c", subcore_axis_name="s")


@functools.partial(
    pl.kernel,
    mesh=_sc_mesh,
    compiler_params=pltpu.CompilerParams(needs_layout_passes=False),
    out_type=jax.ShapeDtypeStruct((B_TOTAL * DIM,), jnp.float32),
    scratch_types=[
        pltpu.VMEM((DIM * VOCAB,), jnp.float32),
        pltpu.VMEM((CHUNK,), jnp.int32),
        pltpu.VMEM((CHUNK,), jnp.int32),
        pltpu.VMEM((ROWS_PER_STEP,), jnp.float32),
        pltpu.VMEM((ROWS_PER_STEP,), jnp.float32),
        pltpu.SemaphoreType.DMA,
        pltpu.SemaphoreType.DMA,
        pltpu.SemaphoreType.DMA,
        pltpu.SemaphoreType.DMA,
    ],
)
def _gather_kernel(table_hbm, idx_hbm, out_hbm, table_v, idx_a, idx_b,
                   rows_a, rows_b, sem_ia, sem_ib, sem_oa, sem_ob):
    wid = lax.axis_index("s") * NUM_CORES + lax.axis_index("c")
    woff = wid * CHUNK
    bufs = ((idx_a, rows_a, sem_ia, sem_oa), (idx_b, rows_b, sem_ib, sem_ob))

    pltpu.sync_copy(table_hbm, table_v)

    def idx_src(i):
        return idx_hbm.at[pl.ds(i * (IH * 1024) + woff, CHUNK)]

    def out_dst(i, jl):
        return out_hbm.at[pl.ds((8 * i + jl) * (IH * 1024) + woff, CHUNK)]

    def compute(idx_v, rows_v):
        def ubody(u, c):
            jl = u // IH_PER_W
            ihh = u % IH_PER_W
            src_base = ihh * 1024 + jl * 128
            dst_base = jl * CHUNK + ihh * 1024
            for g in range(8):
                ids = idx_v[pl.ds(src_base + 16 * g, 16)]
                for d in range(DIM):
                    val = plsc.load_gather(table_v, [ids + d * VOCAB])
                    rows_v[pl.ds(dst_base + d * 128 + 16 * g, 16)] = val
            return c

        lax.fori_loop(0, 8 * IH_PER_W, ubody, 0)

    def do_iter(i):
        idx_v, rows_v, sem_i, sem_o = bufs[i % 2]
        pltpu.make_async_copy(idx_src(i), idx_v, sem_i).wait()
        if i >= 2:
            for jl in range(8):
                pltpu.make_async_copy(
                    rows_v.at[pl.ds(jl * CHUNK, CHUNK)], out_dst(i - 2, jl),
                    sem_o).wait()
        compute(idx_v, rows_v)
        for jl in range(8):
            pltpu.async_copy(
                rows_v.at[pl.ds(jl * CHUNK, CHUNK)], out_dst(i, jl), sem_o)
        if i + 2 < JH:
            pltpu.async_copy(idx_src(i + 2), idx_v, sem_i)

    pltpu.async_copy(idx_src(0), bufs[0][0], bufs[0][2])
    pltpu.async_copy(idx_src(1), bufs[1][0], bufs[1][2])
    for i in range(JH):
        do_iter(i)
    for i in (JH - 2, JH - 1):
        _, rows_v, _, sem_o = bufs[i % 2]
        for jl in range(8):
            pltpu.make_async_copy(
                rows_v.at[pl.ds(jl * CHUNK, CHUNK)], out_dst(i, jl),
                sem_o).wait()


def kernel(input_ids, emb, W, b):
    tableT = _table_call(emb, W, b.reshape(DIM, 1)).reshape(-1)
    idx = (
        input_ids.astype(jnp.int32)
        .reshape(128, 128, JH, 8)
        .transpose(2, 0, 3, 1)
        .reshape(-1)
    )
    out = _gather_kernel(tableT, idx)
    return (
        out.reshape(200, 128, DIM, 128)
        .transpose(1, 3, 0, 2)
        .reshape(input_ids.shape + (DIM,))
    )


# speedup vs baseline: 25.5945x; 2.9736x over previous
import functools

import jax
import jax.numpy as jnp
from jax import lax
from jax.experimental import pallas as pl
from jax.experimental.pallas import tpu as pltpu
from jax.experimental.pallas import tpu_sc as plsc

VOCAB = 100
DIM = 8
B_TOTAL = 16384 * 200

NUM_CORES = 2
NUM_SUBCORES = 16
NUM_WORKERS = NUM_CORES * NUM_SUBCORES
JH = 25
IH = 128
IH_PER_W = IH // NUM_WORKERS
CHUNK = IH_PER_W * 8 * 128
ROWS_PER_STEP = CHUNK * DIM


def _table_body(emb_ref, w_ref, b_ref, table_ref):
    table_ref[...] = (
        jnp.dot(w_ref[...], emb_ref[...].T, preferred_element_type=jnp.float32)
        + b_ref[...]
    )


_table_call = pl.pallas_call(
    _table_body,
    out_shape=jax.ShapeDtypeStruct((DIM, VOCAB), jnp.float32),
)

_sc_mesh = plsc.VectorSubcoreMesh(core_axis_name="c", subcore_axis_name="s")


@functools.partial(
    pl.kernel,
    mesh=_sc_mesh,
    compiler_params=pltpu.CompilerParams(needs_layout_passes=False),
    out_type=jax.ShapeDtypeStruct((B_TOTAL * DIM,), jnp.float32),
    scratch_types=[
        pltpu.VMEM((DIM * VOCAB,), jnp.float32),
        pltpu.VMEM((CHUNK,), jnp.int32),
        pltpu.VMEM((CHUNK,), jnp.int32),
        pltpu.VMEM((ROWS_PER_STEP,), jnp.float32),
        pltpu.VMEM((ROWS_PER_STEP,), jnp.float32),
        pltpu.SemaphoreType.DMA,
        pltpu.SemaphoreType.DMA,
        pltpu.SemaphoreType.DMA,
        pltpu.SemaphoreType.DMA,
    ],
)
def _gather_kernel(table_hbm, idx_hbm, out_hbm, table_v, idx_a, idx_b,
                   rows_a, rows_b, sem_ia, sem_ib, sem_oa, sem_ob):
    wid = lax.axis_index("s") * NUM_CORES + lax.axis_index("c")
    woff = wid * CHUNK
    bufs = ((idx_a, rows_a, sem_ia, sem_oa), (idx_b, rows_b, sem_ib, sem_ob))

    pltpu.sync_copy(table_hbm, table_v)

    def idx_src(i):
        return idx_hbm.at[pl.ds(i * (IH * 1024) + woff, CHUNK)]

    def out_dst(i, jl):
        return out_hbm.at[pl.ds((8 * i + jl) * (IH * 1024) + woff, CHUNK)]

    def compute(idx_v, rows_v):
        @plsc.parallel_loop(0, 8 * IH_PER_W, unroll=2)
        def ubody(u):
            jl = u // IH_PER_W
            ihh = u % IH_PER_W
            src_base = ihh * 1024 + jl * 128
            dst_base = jl * CHUNK + ihh * 1024
            for g in range(8):
                ids = idx_v[pl.ds(src_base + 16 * g, 16)]
                vals = [
                    plsc.load_gather(table_v, [ids + d * VOCAB])
                    for d in range(DIM)
                ]
                for d in range(DIM):
                    rows_v[pl.ds(dst_base + d * 128 + 16 * g, 16)] = vals[d]

    def do_iter(i, b):
        idx_v, rows_v, sem_i, sem_o = bufs[b]
        pltpu.make_async_copy(idx_src(i), idx_v, sem_i).wait()

        @pl.when(i >= 2)
        def _():
            for jl in range(8):
                pltpu.make_async_copy(
                    rows_v.at[pl.ds(jl * CHUNK, CHUNK)], out_dst(i - 2, jl),
                    sem_o).wait()

        compute(idx_v, rows_v)
        for jl in range(8):
            pltpu.async_copy(
                rows_v.at[pl.ds(jl * CHUNK, CHUNK)], out_dst(i, jl), sem_o)

        @pl.when(i + 2 < JH)
        def _():
            pltpu.async_copy(idx_src(i + 2), idx_v, sem_i)

    pltpu.async_copy(idx_src(0), bufs[0][0], bufs[0][2])
    pltpu.async_copy(idx_src(1), bufs[1][0], bufs[1][2])

    def gbody(g, carry):
        do_iter(2 * g, 0)
        do_iter(2 * g + 1, 1)
        return carry

    lax.fori_loop(0, (JH - 1) // 2, gbody, 0)
    do_iter(JH - 1, 0)
    for i, b in ((JH - 2, 1), (JH - 1, 0)):
        _, rows_v, _, sem_o = bufs[b]
        for jl in range(8):
            pltpu.make_async_copy(
                rows_v.at[pl.ds(jl * CHUNK, CHUNK)], out_dst(i, jl),
                sem_o).wait()


def kernel(input_ids, emb, W, b):
    tableT = _table_call(emb, W, b.reshape(DIM, 1)).reshape(-1)
    idx = (
        input_ids.astype(jnp.int32)
        .reshape(128, 128, JH, 8)
        .transpose(2, 0, 3, 1)
        .reshape(-1)
    )
    out = _gather_kernel(tableT, idx)
    return (
        out.reshape(200, 128, DIM, 128)
        .transpose(1, 3, 0, 2)
        .reshape(input_ids.shape + (DIM,))
    )


# speedup vs baseline: 31.4671x; 1.2294x over previous
import functools

import jax
import jax.numpy as jnp
from jax import lax
from jax.experimental import pallas as pl
from jax.experimental.pallas import tpu as pltpu
from jax.experimental.pallas import tpu_sc as plsc

VOCAB = 100
DIM = 8
B_TOTAL = 16384 * 200

NUM_CORES = 2
NUM_SUBCORES = 16
NUM_WORKERS = NUM_CORES * NUM_SUBCORES
JH = 25
IH = 128
IH_PER_W = IH // NUM_WORKERS
CHUNK = IH_PER_W * 8 * 128
ROWS_PER_STEP = CHUNK * DIM


def _table_body(emb_ref, w_ref, b_ref, table_ref):
    table_ref[...] = (
        jnp.dot(w_ref[...], emb_ref[...].T, preferred_element_type=jnp.float32)
        + b_ref[...]
    )


_table_call = pl.pallas_call(
    _table_body,
    out_shape=jax.ShapeDtypeStruct((DIM, VOCAB), jnp.float32),
)

_sc_mesh = plsc.VectorSubcoreMesh(core_axis_name="c", subcore_axis_name="s")


@functools.partial(
    pl.kernel,
    mesh=_sc_mesh,
    compiler_params=pltpu.CompilerParams(needs_layout_passes=False),
    out_type=jax.ShapeDtypeStruct((B_TOTAL * DIM,), jnp.float32),
    scratch_types=[
        pltpu.VMEM((DIM * VOCAB,), jnp.float32),
        pltpu.VMEM((CHUNK,), jnp.int32),
        pltpu.VMEM((CHUNK,), jnp.int32),
        pltpu.VMEM((ROWS_PER_STEP,), jnp.float32),
        pltpu.VMEM((ROWS_PER_STEP,), jnp.float32),
        pltpu.SemaphoreType.DMA,
        pltpu.SemaphoreType.DMA,
        pltpu.SemaphoreType.DMA,
        pltpu.SemaphoreType.DMA,
    ],
)
def _gather_kernel(table_hbm, idx_hbm, out_hbm, table_v, idx_a, idx_b,
                   rows_a, rows_b, sem_ia, sem_ib, sem_oa, sem_ob):
    wid = lax.axis_index("s") * NUM_CORES + lax.axis_index("c")
    woff = wid * CHUNK
    bufs = ((idx_a, rows_a, sem_ia, sem_oa), (idx_b, rows_b, sem_ib, sem_ob))

    pltpu.sync_copy(table_hbm, table_v)

    def idx_src(i):
        return idx_hbm.at[pl.ds(i * (IH * 1024) + woff, CHUNK)]

    def out_dst(i, jl):
        return out_hbm.at[pl.ds((8 * i + jl) * (IH * 1024) + woff, CHUNK)]

    def compute(idx_v, rows_v):
        @plsc.parallel_loop(0, 8 * IH_PER_W, unroll=4)
        def ubody(u):
            jl = u // IH_PER_W
            ihh = u % IH_PER_W
            src_base = ihh * 1024 + jl * 128
            dst_base = jl * CHUNK + ihh * 1024
            for g in range(8):
                ids = idx_v[pl.ds(src_base + 16 * g, 16)]
                vals = [
                    plsc.load_gather(table_v, [ids + d * VOCAB])
                    for d in range(DIM)
                ]
                for d in range(DIM):
                    rows_v[pl.ds(dst_base + d * 128 + 16 * g, 16)] = vals[d]

    def do_iter(i, b):
        idx_v, rows_v, sem_i, sem_o = bufs[b]
        pltpu.make_async_copy(idx_src(i), idx_v, sem_i).wait()

        @pl.when(i >= 2)
        def _():
            for jl in range(8):
                pltpu.make_async_copy(
                    rows_v.at[pl.ds(jl * CHUNK, CHUNK)], out_dst(i - 2, jl),
                    sem_o).wait()

        compute(idx_v, rows_v)
        for jl in range(8):
            pltpu.async_copy(
                rows_v.at[pl.ds(jl * CHUNK, CHUNK)], out_dst(i, jl), sem_o)

        @pl.when(i + 2 < JH)
        def _():
            pltpu.async_copy(idx_src(i + 2), idx_v, sem_i)

    pltpu.async_copy(idx_src(0), bufs[0][0], bufs[0][2])
    pltpu.async_copy(idx_src(1), bufs[1][0], bufs[1][2])

    def gbody(g, carry):
        do_iter(2 * g, 0)
        do_iter(2 * g + 1, 1)
        return carry

    lax.fori_loop(0, (JH - 1) // 2, gbody, 0)
    do_iter(JH - 1, 0)
    for i, b in ((JH - 2, 1), (JH - 1, 0)):
        _, rows_v, _, sem_o = bufs[b]
        for jl in range(8):
            pltpu.make_async_copy(
                rows_v.at[pl.ds(jl * CHUNK, CHUNK)], out_dst(i, jl),
                sem_o).wait()


def kernel(input_ids, emb, W, b):
    tableT = _table_call(emb, W, b.reshape(DIM, 1)).reshape(-1)
    idx = (
        input_ids.astype(jnp.int32)
        .reshape(128, 128, JH, 8)
        .transpose(2, 0, 3, 1)
        .reshape(-1)
    )
    out = _gather_kernel(tableT, idx)
    return (
        out.reshape(200, 128, DIM, 128)
        .transpose(1, 3, 0, 2)
        .reshape(input_ids.shape + (DIM,))
    )
